# baseline (speedup 1.0000x reference)
import functools

import jax
import jax.numpy as jnp
from jax.experimental import pallas as pl


def _body(emb_ref, vis_ref, bbox_ref, kp_ref, mask_ref,
          w1_ref, b1_ref, w2_ref, b2_ref, out_ref, *, T, S, inv_s):
    x = jnp.concatenate(
        [emb_ref[...], vis_ref[...], bbox_ref[...], kp_ref[...]], axis=-1)
    x2 = x.reshape(T * S, x.shape[-1])
    h = jax.lax.dot_general(
        x2, w1_ref[...], (((1,), (1,)), ((), ())),
        preferred_element_type=jnp.float32)
    h = jnp.maximum(h + b1_ref[...], 0.0)
    m = mask_ref[...]
    hm = h.reshape(T, S, h.shape[-1]) * m[..., None]
    hs = hm.sum(axis=1)
    cnt = m.sum(axis=1)
    out = jax.lax.dot_general(
        hs, w2_ref[...], (((1,), (1,)), ((), ())),
        preferred_element_type=jnp.float32)
    out_ref[...] = (out + cnt[:, None] * b2_ref[...]) * inv_s


def kernel(embeddings, visibility_scores, bbox_ltwh, keypoints_xyc,
           feats_masks, W1, b1, W2, b2):
    B, N, S, E = embeddings.shape
    KP = keypoints_xyc.shape[3]
    M = B * N
    F = W1.shape[1]
    TOK = W2.shape[0]

    T = 16

    emb = embeddings.reshape(M, S, E)
    vis = visibility_scores.reshape(M, S, 1)
    bbox = bbox_ltwh.reshape(M, S, 4)
    kp = keypoints_xyc.reshape(M, S, KP * 3)
    maskf = feats_masks.reshape(M, S).astype(jnp.float32)
    b1r = b1.reshape(1, F)
    b2r = b2.reshape(1, TOK)

    grid = (M // T,)
    body = functools.partial(_body, T=T, S=S, inv_s=1.0 / S)
    out = pl.pallas_call(
        body,
        grid=grid,
        in_specs=[
            pl.BlockSpec((T, S, E), lambda i: (i, 0, 0)),
            pl.BlockSpec((T, S, 1), lambda i: (i, 0, 0)),
            pl.BlockSpec((T, S, 4), lambda i: (i, 0, 0)),
            pl.BlockSpec((T, S, KP * 3), lambda i: (i, 0, 0)),
            pl.BlockSpec((T, S), lambda i: (i, 0)),
            pl.BlockSpec((F, F), lambda i: (0, 0)),
            pl.BlockSpec((1, F), lambda i: (0, 0)),
            pl.BlockSpec((TOK, F), lambda i: (0, 0)),
            pl.BlockSpec((1, TOK), lambda i: (0, 0)),
        ],
        out_specs=pl.BlockSpec((T, TOK), lambda i: (i, 0)),
        out_shape=jax.ShapeDtypeStruct((M, TOK), jnp.float32),
    )(emb, vis, bbox, kp, maskf, W1, b1r, W2, b2r)
    return out.reshape(B, N, TOK)
